# Initial kernel scaffold; baseline (speedup 1.0000x reference)
import jax
import jax.numpy as jnp
from jax.experimental import pallas as pl


def kernel(features, edge_index, W, b):
    raise NotImplementedError("write your pallas kernel here")


# speedup vs baseline: 5.7681x; 5.7681x over previous
import functools

import jax
import jax.numpy as jnp
from jax import lax
from jax.experimental import pallas as pl
from jax.experimental.pallas import tpu as pltpu
from jax.experimental.pallas import tpu_sc as plsc

N_NODES = 10000
N_EDGES = 320000
D = 128
NC = 2
NS = 16
NW = NC * NS
SUB = 128
NCHUNK = N_EDGES // SUB
ITERS = -(-NCHUNK // NW)
NPAD = 10240
ROWS_PT = N_NODES // NS

_mesh = plsc.VectorSubcoreMesh(
    core_axis_name="c", subcore_axis_name="s", num_cores=NC, num_subcores=NS)


@functools.partial(
    pl.kernel,
    out_type=jax.ShapeDtypeStruct((NC, 2, NPAD), jnp.float32),
    mesh=_mesh,
    scratch_types=[
        pltpu.VMEM((SUB,), jnp.int32),
        pltpu.VMEM((SUB,), jnp.float32),
        pltpu.VMEM((640,), jnp.float32),
        pltpu.VMEM_SHARED((NPAD,), jnp.float32),
        pltpu.VMEM_SHARED((NPAD,), jnp.float32),
    ],
)
def _degree_kernel(src_hbm, dst_hbm, out_hbm, idx_v, ones_v, zeros_v,
                   hist_s, hist_d):
    c = lax.axis_index("c")
    s = lax.axis_index("s")
    w = c * NS + s

    def fill_zeros(i, _):
        zeros_v[pl.ds(i * 16, 16)] = jnp.zeros((16,), jnp.float32)
        return 0
    lax.fori_loop(0, 640 // 16, fill_zeros, 0)

    def fill_ones(i, _):
        ones_v[pl.ds(i * 16, 16)] = jnp.ones((16,), jnp.float32)
        return 0
    lax.fori_loop(0, SUB // 16, fill_ones, 0)

    pltpu.sync_copy(zeros_v, hist_s.at[pl.ds(s * 640, 640)])
    pltpu.sync_copy(zeros_v, hist_d.at[pl.ds(s * 640, 640)])
    plsc.subcore_barrier()

    def body(i, _):
        j = i * NW + w

        @pl.when(j < NCHUNK)
        def _():
            pltpu.sync_copy(src_hbm.at[pl.ds(j * SUB, SUB)], idx_v)
            pltpu.sync_copy(ones_v, hist_s.at[idx_v], add=True)
            pltpu.sync_copy(dst_hbm.at[pl.ds(j * SUB, SUB)], idx_v)
            pltpu.sync_copy(ones_v, hist_d.at[idx_v], add=True)
        return 0
    lax.fori_loop(0, ITERS, body, 0)
    plsc.subcore_barrier()

    @pl.when(s == 0)
    def _():
        pltpu.sync_copy(hist_s, out_hbm.at[c, 0])

    @pl.when(s == 1)
    def _():
        pltpu.sync_copy(hist_d, out_hbm.at[c, 1])


@functools.partial(
    pl.kernel,
    out_type=jax.ShapeDtypeStruct((NC, N_NODES, D), jnp.float32),
    mesh=_mesh,
    scratch_types=[
        pltpu.VMEM((SUB,), jnp.int32),
        pltpu.VMEM((SUB,), jnp.int32),
        pltpu.VMEM((SUB, D), jnp.float32),
        pltpu.VMEM((8, D), jnp.float32),
        pltpu.VMEM_SHARED((N_NODES, D), jnp.float32),
        pltpu.SemaphoreType.DMA,
    ],
)
def _agg_kernel(y_hbm, src_hbm, dst_hbm, out_hbm, sidx, didx, rows, zrows,
                agg, sem):
    c = lax.axis_index("c")
    s = lax.axis_index("s")
    w = c * NS + s

    def fill_zrows(i, _):
        zrows[i // 8, pl.ds((i % 8) * 16, 16)] = jnp.zeros((16,), jnp.float32)
        return 0
    lax.fori_loop(0, 8 * 8, fill_zrows, 0)

    NRCHUNK = N_NODES // 8
    RITERS = -(-NRCHUNK // NS)

    def zero_agg(i, _):
        j = i * NS + s

        @pl.when(j < NRCHUNK)
        def _():
            pltpu.sync_copy(zrows, agg.at[pl.ds(j * 8, 8)])
        return 0
    lax.fori_loop(0, RITERS, zero_agg, 0)
    plsc.subcore_barrier()

    def body(i, _):
        j = i * NW + w

        @pl.when(j < NCHUNK)
        def _():
            pltpu.sync_copy(src_hbm.at[pl.ds(j * SUB, SUB)], sidx)
            pltpu.sync_copy(dst_hbm.at[pl.ds(j * SUB, SUB)], didx)
            pltpu.async_copy(y_hbm.at[sidx], rows, sem).wait()
            pltpu.sync_copy(rows, agg.at[didx], add=True)
        return 0
    lax.fori_loop(0, ITERS, body, 0)
    plsc.subcore_barrier()

    def copy_out(i, _):
        j = i * NS + s

        @pl.when(j < NRCHUNK)
        def _():
            pltpu.sync_copy(agg.at[pl.ds(j * 8, 8)],
                            out_hbm.at[c, pl.ds(j * 8, 8)])
        return 0
    lax.fori_loop(0, RITERS, copy_out, 0)


_RB = 2048


def _prescale_matmul_body(deg_ref, x_ref, w_ref, y_ref):
    d = deg_ref[0, 0, :] + deg_ref[1, 0, :]
    ns = lax.rsqrt(jnp.maximum(d, 1.0))
    y_ref[...] = jnp.dot(x_ref[...] * ns[:, None], w_ref[...],
                         preferred_element_type=jnp.float32)


def _finish_body(deg_ref, b_ref, p_ref, o_ref):
    d = deg_ref[0, 1, :] + deg_ref[1, 1, :]
    nd = lax.rsqrt(jnp.maximum(d, 1.0))
    o_ref[...] = (p_ref[0] + p_ref[1]) * nd[:, None] + b_ref[...]


def kernel(features, edge_index, W, b):
    edge_index = edge_index.astype(jnp.int32)
    src = edge_index[0]
    dst = edge_index[1]

    deg = _degree_kernel(src, dst)

    y = pl.pallas_call(
        _prescale_matmul_body,
        grid=(pl.cdiv(N_NODES, _RB),),
        in_specs=[
            pl.BlockSpec((NC, 2, _RB), lambda i: (0, 0, i)),
            pl.BlockSpec((_RB, D), lambda i: (i, 0)),
            pl.BlockSpec((D, D), lambda i: (0, 0)),
        ],
        out_specs=pl.BlockSpec((_RB, D), lambda i: (i, 0)),
        out_shape=jax.ShapeDtypeStruct((N_NODES, D), jnp.float32),
    )(deg, features, W)

    parts = _agg_kernel(y, src, dst)

    out = pl.pallas_call(
        _finish_body,
        grid=(pl.cdiv(N_NODES, _RB),),
        in_specs=[
            pl.BlockSpec((NC, 2, _RB), lambda i: (0, 0, i)),
            pl.BlockSpec((1, D), lambda i: (0, 0)),
            pl.BlockSpec((NC, _RB, D), lambda i: (0, i, 0)),
        ],
        out_specs=pl.BlockSpec((_RB, D), lambda i: (i, 0)),
        out_shape=jax.ShapeDtypeStruct((N_NODES, D), jnp.float32),
    )(deg, b.reshape(1, D), parts)

    return out
